# baseline (speedup 1.0000x reference)
import functools

import jax
import jax.numpy as jnp
from jax import lax
from jax.experimental import pallas as pl
from jax.experimental.pallas import tpu as pltpu
from jax.experimental.pallas import tpu_sc as plsc

_INFO = plsc.get_sparse_core_info()
_NC = _INFO.num_cores
_NS = _INFO.num_subcores
_NW = _NC * _NS
_CHUNK = 64
_NBUF = 3


def _make_gather(B, V, D):
  b_per_w = B // _NW
  n_chunks = b_per_w // _CHUNK
  mesh = plsc.VectorSubcoreMesh(core_axis_name="c", subcore_axis_name="s")

  @functools.partial(
      pl.kernel,
      mesh=mesh,
      out_type=jax.ShapeDtypeStruct((B, D), jnp.float32),
      scratch_types=[
          pltpu.VMEM((n_chunks, _CHUNK), jnp.int32),
          pltpu.VMEM((_NBUF, _CHUNK, D), jnp.float32),
          pltpu.SemaphoreType.DMA((_NBUF,)),
          pltpu.SemaphoreType.DMA((_NBUF,)),
      ],
  )
  def gather_kernel(t_hbm, table_hbm, out_hbm, idx_v, rows_v, gsem, wsem):
    wid = lax.axis_index("s") * _NC + lax.axis_index("c")
    base = wid * b_per_w
    for c in range(n_chunks):
      pltpu.sync_copy(t_hbm.at[pl.ds(base + c * _CHUNK, _CHUNK)], idx_v.at[c])
    for c in range(min(_NBUF, n_chunks)):
      pltpu.async_copy(table_hbm.at[idx_v.at[c]], rows_v.at[c], gsem.at[c])
    for c in range(n_chunks):
      b = c % _NBUF
      out_slc = out_hbm.at[pl.ds(base + c * _CHUNK, _CHUNK)]
      pltpu.make_async_copy(table_hbm.at[idx_v.at[c]], rows_v.at[b],
                            gsem.at[b]).wait()
      pltpu.async_copy(rows_v.at[b], out_slc, wsem.at[b])
      nxt = c + _NBUF
      if nxt < n_chunks:
        pltpu.make_async_copy(rows_v.at[b], out_slc, wsem.at[b]).wait()
        pltpu.async_copy(table_hbm.at[idx_v.at[nxt]], rows_v.at[b],
                         gsem.at[b])
    for c in range(max(n_chunks - _NBUF, 0), n_chunks):
      b = c % _NBUF
      out_slc = out_hbm.at[pl.ds(base + c * _CHUNK, _CHUNK)]
      pltpu.make_async_copy(rows_v.at[b], out_slc, wsem.at[b]).wait()

  return gather_kernel


def kernel(t, table):
  B = t.shape[0]
  V, D = table.shape
  out = _make_gather(B, V, D)(t.astype(jnp.int32), table)
  return out[:, :, None, None]


# speedup vs baseline: 1.5198x; 1.5198x over previous
import functools

import jax
import jax.numpy as jnp
from jax import lax
from jax.experimental import pallas as pl
from jax.experimental.pallas import tpu as pltpu
from jax.experimental.pallas import tpu_sc as plsc

_INFO = plsc.get_sparse_core_info()
_NC = _INFO.num_cores
_NS = _INFO.num_subcores
_NW = _NC * _NS
_CHUNK = 64
_NBUF = 3


def _make_gather(B, V, D):
  b_per_w = B // _NW
  n_chunks = b_per_w // _CHUNK
  mesh = plsc.VectorSubcoreMesh(core_axis_name="c", subcore_axis_name="s")

  @functools.partial(
      pl.kernel,
      mesh=mesh,
      out_type=jax.ShapeDtypeStruct((B, 1, 1, D), jnp.float32),
      scratch_types=[
          pltpu.VMEM((n_chunks, _CHUNK), jnp.int32),
          pltpu.VMEM((_NBUF, _CHUNK, 1, 1, D), jnp.float32),
          pltpu.SemaphoreType.DMA((_NBUF,)),
          pltpu.SemaphoreType.DMA((_NBUF,)),
      ],
  )
  def gather_kernel(t_hbm, table_hbm, out_hbm, idx_v, rows_v, gsem, wsem):
    wid = lax.axis_index("s") * _NC + lax.axis_index("c")
    base = wid * b_per_w
    for c in range(n_chunks):
      pltpu.sync_copy(t_hbm.at[pl.ds(base + c * _CHUNK, _CHUNK)], idx_v.at[c])
    for c in range(min(_NBUF, n_chunks)):
      pltpu.async_copy(table_hbm.at[idx_v.at[c]], rows_v.at[c], gsem.at[c])
    for c in range(n_chunks):
      b = c % _NBUF
      out_slc = out_hbm.at[pl.ds(base + c * _CHUNK, _CHUNK)]
      pltpu.make_async_copy(table_hbm.at[idx_v.at[c]], rows_v.at[b],
                            gsem.at[b]).wait()
      pltpu.async_copy(rows_v.at[b], out_slc, wsem.at[b])
      nxt = c + _NBUF
      if nxt < n_chunks:
        pltpu.make_async_copy(rows_v.at[b], out_slc, wsem.at[b]).wait()
        pltpu.async_copy(table_hbm.at[idx_v.at[nxt]], rows_v.at[b],
                         gsem.at[b])
    for c in range(max(n_chunks - _NBUF, 0), n_chunks):
      b = c % _NBUF
      out_slc = out_hbm.at[pl.ds(base + c * _CHUNK, _CHUNK)]
      pltpu.make_async_copy(rows_v.at[b], out_slc, wsem.at[b]).wait()

  return gather_kernel


def kernel(t, table):
  B = t.shape[0]
  V, D = table.shape
  out = _make_gather(B, V, D)(t.astype(jnp.int32), table[:, None, None, :])
  return out.reshape(B, D, 1, 1)


# speedup vs baseline: 1.7321x; 1.1397x over previous
import functools

import jax
import jax.numpy as jnp
from jax import lax
from jax.experimental import pallas as pl
from jax.experimental.pallas import tpu as pltpu
from jax.experimental.pallas import tpu_sc as plsc

_INFO = plsc.get_sparse_core_info()
_NC = _INFO.num_cores
_NS = _INFO.num_subcores
_NW = _NC * _NS
_CHUNK = 32
_NBUF = 4


def _make_gather(B, V, D):
  b_per_w = B // _NW
  n_chunks = b_per_w // _CHUNK
  stage = -(-V // _NS)
  stage_last = V - stage * (_NS - 1)
  mesh = plsc.VectorSubcoreMesh(core_axis_name="c", subcore_axis_name="s")

  @functools.partial(
      pl.kernel,
      mesh=mesh,
      out_type=jax.ShapeDtypeStruct((B, 1, 1, D), jnp.float32),
      scratch_types=[
          pltpu.VMEM_SHARED((V, 1, 1, D), jnp.float32),
          pltpu.VMEM((n_chunks, _CHUNK), jnp.int32),
          pltpu.VMEM((_NBUF, _CHUNK, 1, 1, D), jnp.float32),
          pltpu.SemaphoreType.DMA((_NBUF,)),
          pltpu.SemaphoreType.DMA((_NBUF,)),
      ],
  )
  def gather_kernel(t_hbm, table_hbm, out_hbm, table_sp, idx_v, rows_v,
                    gsem, wsem):
    cid = lax.axis_index("c")
    sid = lax.axis_index("s")
    wid = sid * _NC + cid
    base = wid * b_per_w

    @pl.when(sid < _NS - 1)
    def _stage():
      pltpu.sync_copy(table_hbm.at[pl.ds(sid * stage, stage)],
                      table_sp.at[pl.ds(sid * stage, stage)])

    @pl.when(sid == _NS - 1)
    def _stage_tail():
      pltpu.sync_copy(table_hbm.at[pl.ds((_NS - 1) * stage, stage_last)],
                      table_sp.at[pl.ds((_NS - 1) * stage, stage_last)])

    for c in range(n_chunks):
      pltpu.sync_copy(t_hbm.at[pl.ds(base + c * _CHUNK, _CHUNK)], idx_v.at[c])
    plsc.subcore_barrier()
    for c in range(min(_NBUF, n_chunks)):
      pltpu.async_copy(table_sp.at[idx_v.at[c]], rows_v.at[c], gsem.at[c])
    for c in range(n_chunks):
      b = c % _NBUF
      out_slc = out_hbm.at[pl.ds(base + c * _CHUNK, _CHUNK)]
      pltpu.make_async_copy(table_sp.at[idx_v.at[c]], rows_v.at[b],
                            gsem.at[b]).wait()
      pltpu.async_copy(rows_v.at[b], out_slc, wsem.at[b])
      nxt = c + _NBUF
      if nxt < n_chunks:
        pltpu.make_async_copy(rows_v.at[b], out_slc, wsem.at[b]).wait()
        pltpu.async_copy(table_sp.at[idx_v.at[nxt]], rows_v.at[b],
                         gsem.at[b])
    for c in range(max(n_chunks - _NBUF, 0), n_chunks):
      b = c % _NBUF
      out_slc = out_hbm.at[pl.ds(base + c * _CHUNK, _CHUNK)]
      pltpu.make_async_copy(rows_v.at[b], out_slc, wsem.at[b]).wait()

  return gather_kernel


def kernel(t, table):
  B = t.shape[0]
  V, D = table.shape
  out = _make_gather(B, V, D)(t.astype(jnp.int32), table[:, None, None, :])
  return out.reshape(B, D, 1, 1)


# speedup vs baseline: 1.9992x; 1.1542x over previous
import functools

import jax
import jax.numpy as jnp
from jax import lax
from jax.experimental import pallas as pl
from jax.experimental.pallas import tpu as pltpu
from jax.experimental.pallas import tpu_sc as plsc

_INFO = plsc.get_sparse_core_info()
_NC = _INFO.num_cores
_NS = _INFO.num_subcores
_NW = _NC * _NS
_CHUNK = 32
_NBUF = 4


def _make_gather(B, V, D):
  b_per_w = B // _NW
  n_chunks = b_per_w // _CHUNK
  n_groups = n_chunks // _NBUF
  stage = -(-V // _NS)
  stage_last = V - stage * (_NS - 1)
  mesh = plsc.VectorSubcoreMesh(core_axis_name="c", subcore_axis_name="s")

  @functools.partial(
      pl.kernel,
      mesh=mesh,
      out_type=jax.ShapeDtypeStruct((B, 1, 1, D), jnp.float32),
      scratch_types=[
          pltpu.VMEM_SHARED((V, 1, 1, D), jnp.float32),
          pltpu.VMEM((b_per_w,), jnp.int32),
          pltpu.VMEM((_NBUF, _CHUNK, 1, 1, D), jnp.float32),
          pltpu.SemaphoreType.DMA((_NBUF,)),
          pltpu.SemaphoreType.DMA((_NBUF,)),
          pltpu.SemaphoreType.DMA,
      ],
  )
  def gather_kernel(t_hbm, table_hbm, out_hbm, table_sp, idx_v, rows_v,
                    gsem, wsem, ssem):
    cid = lax.axis_index("c")
    sid = lax.axis_index("s")
    wid = sid * _NC + cid
    base = wid * b_per_w

    pltpu.async_copy(t_hbm.at[pl.ds(base, b_per_w)], idx_v, ssem)

    @pl.when(sid < _NS - 1)
    def _stage():
      pltpu.sync_copy(table_hbm.at[pl.ds(sid * stage, stage)],
                      table_sp.at[pl.ds(sid * stage, stage)])

    @pl.when(sid == _NS - 1)
    def _stage_tail():
      pltpu.sync_copy(table_hbm.at[pl.ds((_NS - 1) * stage, stage_last)],
                      table_sp.at[pl.ds((_NS - 1) * stage, stage_last)])

    pltpu.make_async_copy(t_hbm.at[pl.ds(base, b_per_w)], idx_v, ssem).wait()
    plsc.subcore_barrier()

    def chunk_idx(c):
      return idx_v.at[pl.ds(c * _CHUNK, _CHUNK)]

    for b in range(_NBUF):
      pltpu.async_copy(table_sp.at[chunk_idx(b)], rows_v.at[b], gsem.at[b])

    def group(g, _):
      for b in range(_NBUF):
        c = g * _NBUF + b
        out_slc = out_hbm.at[pl.ds(base + c * _CHUNK, _CHUNK)]
        pltpu.make_async_copy(table_sp.at[chunk_idx(c)], rows_v.at[b],
                              gsem.at[b]).wait()
        pltpu.async_copy(rows_v.at[b], out_slc, wsem.at[b])
        pltpu.make_async_copy(rows_v.at[b], out_slc, wsem.at[b]).wait()

        @pl.when(g < n_groups - 1)
        def _next():
          pltpu.async_copy(table_sp.at[chunk_idx(c + _NBUF)], rows_v.at[b],
                           gsem.at[b])

    lax.fori_loop(0, n_groups, group, None, unroll=False)

  return gather_kernel


def kernel(t, table):
  B = t.shape[0]
  V, D = table.shape
  out = _make_gather(B, V, D)(t.astype(jnp.int32), table[:, None, None, :])
  return out.reshape(B, D, 1, 1)


# speedup vs baseline: 2.0648x; 1.0328x over previous
import functools

import jax
import jax.numpy as jnp
from jax import lax
from jax.experimental import pallas as pl
from jax.experimental.pallas import tpu as pltpu
from jax.experimental.pallas import tpu_sc as plsc

_INFO = plsc.get_sparse_core_info()
_NC = _INFO.num_cores
_NS = _INFO.num_subcores
_NW = _NC * _NS
_CHUNK = 32
_NBUF = 4


def _make_gather(B, V, D):
  b_per_w = B // _NW
  n_chunks = b_per_w // _CHUNK
  n_groups = n_chunks // _NBUF
  stage = -(-V // _NS)
  stage_last = V - stage * (_NS - 1)
  mesh = plsc.VectorSubcoreMesh(core_axis_name="c", subcore_axis_name="s")

  @functools.partial(
      pl.kernel,
      mesh=mesh,
      out_type=jax.ShapeDtypeStruct((B, 1, 1, D), jnp.float32),
      scratch_types=[
          pltpu.VMEM_SHARED((V, 1, 1, D), jnp.float32),
          pltpu.VMEM((b_per_w,), jnp.int32),
          pltpu.VMEM((_NBUF, _CHUNK, 1, 1, D), jnp.float32),
          pltpu.SemaphoreType.DMA((_NBUF,)),
          pltpu.SemaphoreType.DMA((_NBUF,)),
          pltpu.SemaphoreType.DMA,
      ],
  )
  def gather_kernel(t_hbm, table_hbm, out_hbm, table_sp, idx_v, rows_v,
                    gsem, wsem, ssem):
    cid = lax.axis_index("c")
    sid = lax.axis_index("s")
    wid = sid * _NC + cid
    base = wid * b_per_w

    pltpu.async_copy(t_hbm.at[pl.ds(base, b_per_w)], idx_v, ssem)

    @pl.when(sid < _NS - 1)
    def _stage():
      pltpu.sync_copy(table_hbm.at[pl.ds(sid * stage, stage)],
                      table_sp.at[pl.ds(sid * stage, stage)])

    @pl.when(sid == _NS - 1)
    def _stage_tail():
      pltpu.sync_copy(table_hbm.at[pl.ds((_NS - 1) * stage, stage_last)],
                      table_sp.at[pl.ds((_NS - 1) * stage, stage_last)])

    pltpu.make_async_copy(t_hbm.at[pl.ds(base, b_per_w)], idx_v, ssem).wait()
    plsc.subcore_barrier()

    def chunk_idx(c):
      return idx_v.at[pl.ds(c * _CHUNK, _CHUNK)]

    for b in range(_NBUF):
      pltpu.async_copy(table_sp.at[chunk_idx(b)], rows_v.at[b], gsem.at[b])

    def group(g, _):
      for b in range(_NBUF):
        c = g * _NBUF + b
        out_slc = out_hbm.at[pl.ds(base + c * _CHUNK, _CHUNK)]
        pltpu.make_async_copy(table_sp.at[chunk_idx(c)], rows_v.at[b],
                              gsem.at[b]).wait()
        pltpu.async_copy(rows_v.at[b], out_slc, wsem.at[b])
      for b in range(_NBUF):
        c = g * _NBUF + b
        out_slc = out_hbm.at[pl.ds(base + c * _CHUNK, _CHUNK)]
        pltpu.make_async_copy(rows_v.at[b], out_slc, wsem.at[b]).wait()

        @pl.when(g < n_groups - 1)
        def _next():
          pltpu.async_copy(table_sp.at[chunk_idx(c + _NBUF)], rows_v.at[b],
                           gsem.at[b])

    lax.fori_loop(0, n_groups, group, None, unroll=False)

  return gather_kernel


def kernel(t, table):
  B = t.shape[0]
  V, D = table.shape
  out = _make_gather(B, V, D)(t.astype(jnp.int32), table[:, None, None, :])
  return out.reshape(B, D, 1, 1)
